# Initial kernel scaffold; baseline (speedup 1.0000x reference)
import jax
import jax.numpy as jnp
from jax.experimental import pallas as pl


def kernel(features, edge_index, W1, b1, W2, b2, W3, b3):
    raise NotImplementedError("write your pallas kernel here")


# speedup vs baseline: 2.7682x; 2.7682x over previous
import functools

import jax
import jax.numpy as jnp
from jax import lax
from jax.experimental import pallas as pl
from jax.experimental.pallas import tpu as pltpu
from jax.experimental.pallas import tpu_sc as plsc

N_NODES = 10000
IN_DIM = 128
HID_DIM = 256
OUT_DIM = 128

NPAD = 10240
N_EDGES2 = 640000
N_TILES = 32
CHUNK = 128
N_CHUNKS = 160
EDGES_PER_TILE = N_CHUNKS * CHUNK
E2_PAD = EDGES_PER_TILE * N_TILES
ROWS_PER_TILE = NPAD // 16

_MESH = dict(core_axis_name="c", subcore_axis_name="s", num_cores=2,
             num_subcores=16)


def _sc_agg_body(table, srcr, dstr, zrows,
                 s_out,
                 src_idx, dst_idx, rows_v, s_sh, sem):
    c = lax.axis_index("c")
    s = lax.axis_index("s")
    wid = s * 2 + c

    pltpu.sync_copy(zrows, s_sh.at[pl.ds(s * ROWS_PER_TILE, ROWS_PER_TILE)])
    plsc.subcore_barrier()

    def chunk(j, carry):
        pltpu.sync_copy(srcr.at[wid].at[j], src_idx)
        pltpu.sync_copy(dstr.at[wid].at[j], dst_idx)
        pltpu.async_copy(table.at[src_idx], rows_v, sem).wait()
        pltpu.sync_copy(rows_v, s_sh.at[dst_idx], add=True)
        return carry

    lax.fori_loop(0, N_CHUNKS, chunk, 0)
    plsc.subcore_barrier()

    rs = pl.ds(s * ROWS_PER_TILE, ROWS_PER_TILE)
    pltpu.sync_copy(s_sh.at[rs], s_out.at[c].at[rs])


def _sc_deg_body(dstr, zrows, ones128,
                 deg_out,
                 dst_idx, ones_v, deg_sh, sem):
    del sem
    c = lax.axis_index("c")
    s = lax.axis_index("s")
    wid = s * 2 + c

    pltpu.sync_copy(zrows, deg_sh.at[pl.ds(s * ROWS_PER_TILE, ROWS_PER_TILE)])
    pltpu.sync_copy(ones128, ones_v)
    plsc.subcore_barrier()

    def chunk(j, carry):
        pltpu.sync_copy(dstr.at[wid].at[j], dst_idx)
        pltpu.sync_copy(ones_v, deg_sh.at[dst_idx], add=True)
        return carry

    lax.fori_loop(0, N_CHUNKS, chunk, 0)
    plsc.subcore_barrier()

    rs = pl.ds(s * ROWS_PER_TILE, ROWS_PER_TILE)
    pltpu.sync_copy(deg_sh.at[rs], deg_out.at[c].at[rs])


_sc_agg = pl.kernel(
    _sc_agg_body,
    out_type=(jax.ShapeDtypeStruct((2, NPAD, 128), jnp.float32),),
    mesh=plsc.VectorSubcoreMesh(**_MESH),
    scratch_types=(
        pltpu.VMEM((CHUNK,), jnp.int32),
        pltpu.VMEM((CHUNK,), jnp.int32),
        pltpu.VMEM((CHUNK, 128), jnp.float32),
        pltpu.VMEM_SHARED((NPAD, 128), jnp.float32),
        pltpu.SemaphoreType.DMA,
    ),
)

_sc_deg = pl.kernel(
    _sc_deg_body,
    out_type=(jax.ShapeDtypeStruct((2, NPAD, 128), jnp.float32),),
    mesh=plsc.VectorSubcoreMesh(**_MESH),
    scratch_types=(
        pltpu.VMEM((CHUNK,), jnp.int32),
        pltpu.VMEM((CHUNK, 128), jnp.float32),
        pltpu.VMEM_SHARED((NPAD, 128), jnp.float32),
        pltpu.SemaphoreType.DMA,
    ),
)


_R = 1000


def _deg_factors(dref):
    d = dref[0][:, 0:1] + dref[1][:, 0:1]
    invd = 1.0 / jnp.maximum(d, 1.0)
    sself = 0.5 * d * invd
    hinv = 0.5 * invd
    hasdeg = (d > 0.0).astype(jnp.float32)
    return sself, hinv, hasdeg


def _t1_body(f, sa, dref, w, b, o):
    sself, hinv, hasdeg = _deg_factors(dref)
    p = sself * f[...] + hinv * (sa[0] + sa[1])
    o[...] = jax.nn.relu(
        jnp.dot(p, w[...], preferred_element_type=jnp.float32) + hasdeg * b[...])


def _t2_body(h, w, b, o):
    o[...] = jnp.dot(h[...], w[...], preferred_element_type=jnp.float32) + b[...]


def _t3_body(z, sb, dref, w, b, h2, z3):
    sself, hinv, _ = _deg_factors(dref)
    h = jax.nn.relu(sself * z[...] + hinv * (sb[0] + sb[1]))
    h2[...] = h
    z3[...] = jnp.dot(h, w[...], preferred_element_type=jnp.float32) + b[...]


def _t4_body(z, sc, dref, o):
    sself, hinv, _ = _deg_factors(dref)
    o[...] = jax.nn.relu(sself * z[...] + hinv * (sc[0] + sc[1]))


def _row_spec(cols):
    return pl.BlockSpec((_R, cols), lambda i: (i, 0))


def _part_spec(cols):
    return pl.BlockSpec((2, _R, cols), lambda i: (0, i, 0))


def _full_spec(r, c):
    return pl.BlockSpec((r, c), lambda i: (0, 0))


def _t1(f, sa, degp, w1, b1):
    return pl.pallas_call(
        _t1_body,
        grid=(N_NODES // _R,),
        in_specs=[_row_spec(IN_DIM), _part_spec(128), _part_spec(128),
                  _full_spec(IN_DIM, HID_DIM), _full_spec(1, HID_DIM)],
        out_specs=_row_spec(HID_DIM),
        out_shape=jax.ShapeDtypeStruct((N_NODES, HID_DIM), jnp.float32),
    )(f, sa, degp, w1, b1)


def _t2(h, w2, b2):
    return pl.pallas_call(
        _t2_body,
        grid=(N_NODES // _R,),
        in_specs=[_row_spec(HID_DIM), _full_spec(HID_DIM, OUT_DIM),
                  _full_spec(1, OUT_DIM)],
        out_specs=_row_spec(OUT_DIM),
        out_shape=jax.ShapeDtypeStruct((N_NODES, OUT_DIM), jnp.float32),
    )(h, w2, b2)


def _t3(z2, sb, degp, w3, b3):
    return pl.pallas_call(
        _t3_body,
        grid=(N_NODES // _R,),
        in_specs=[_row_spec(OUT_DIM), _part_spec(128), _part_spec(128),
                  _full_spec(OUT_DIM, OUT_DIM), _full_spec(1, OUT_DIM)],
        out_specs=[_row_spec(OUT_DIM), _row_spec(OUT_DIM)],
        out_shape=[jax.ShapeDtypeStruct((N_NODES, OUT_DIM), jnp.float32),
                   jax.ShapeDtypeStruct((N_NODES, OUT_DIM), jnp.float32)],
    )(z2, sb, degp, w3, b3)


def _t4(z3, sc, degp):
    return pl.pallas_call(
        _t4_body,
        grid=(N_NODES // _R,),
        in_specs=[_row_spec(OUT_DIM), _part_spec(128), _part_spec(128)],
        out_specs=_row_spec(OUT_DIM),
        out_shape=jax.ShapeDtypeStruct((N_NODES, OUT_DIM), jnp.float32),
    )(z3, sc, degp)


def kernel(features, edge_index, W1, b1, W2, b2, W3, b3):
    src = edge_index[0].astype(jnp.int32)
    dst = edge_index[1].astype(jnp.int32)
    src2 = jnp.concatenate([src, dst])
    dst2 = jnp.concatenate([dst, src])
    pad = E2_PAD - N_EDGES2
    srcr = jnp.pad(src2, (0, pad)).reshape(N_TILES, N_CHUNKS, CHUNK)
    dstr = jnp.pad(dst2, (0, pad), constant_values=NPAD - 1) \
              .reshape(N_TILES, N_CHUNKS, CHUNK)

    zrows = jnp.zeros((ROWS_PER_TILE, 128), jnp.float32)
    ones128 = jnp.ones((CHUNK, 128), jnp.float32)

    b1r = b1.reshape(1, HID_DIM)
    b2r = b2.reshape(1, OUT_DIM)
    b3r = b3.reshape(1, OUT_DIM)

    (degp,) = _sc_deg(dstr, zrows, ones128)
    (sa,) = _sc_agg(features, srcr, dstr, zrows)
    h1 = _t1(features, sa, degp, W1, b1r)
    z2 = _t2(h1, W2, b2r)
    (sb,) = _sc_agg(z2, srcr, dstr, zrows)
    h2, z3 = _t3(z2, sb, degp, W3, b3r)
    (sc,) = _sc_agg(z3, srcr, dstr, zrows)
    logits = _t4(z3, sc, degp)
    return (h2, logits)


# speedup vs baseline: 3.4372x; 1.2417x over previous
import functools

import jax
import jax.numpy as jnp
from jax import lax
from jax.experimental import pallas as pl
from jax.experimental.pallas import tpu as pltpu
from jax.experimental.pallas import tpu_sc as plsc

N_NODES = 10000
IN_DIM = 128
HID_DIM = 256
OUT_DIM = 128

NPAD = 10240
N_EDGES2 = 640000
N_TILES = 32
CHUNK = 128
BLK_CH = 40
N_BLKS = 4
N_CHUNKS = BLK_CH * N_BLKS
EDGES_PER_TILE = N_CHUNKS * CHUNK
E2_PAD = EDGES_PER_TILE * N_TILES
ROWS_PER_TILE = NPAD // 16

_MESH = dict(core_axis_name="c", subcore_axis_name="s", num_cores=2,
             num_subcores=16)


def _sc_agg_body(table, srcr, dstr, zrows,
                 s_out,
                 sidx, didx, rows0, rows1, s_sh, sem0, sem1):
    c = lax.axis_index("c")
    s = lax.axis_index("s")
    wid = s * 2 + c

    pltpu.sync_copy(zrows, s_sh.at[pl.ds(s * ROWS_PER_TILE, ROWS_PER_TILE)])
    plsc.subcore_barrier()

    def block(b, carry):
        pltpu.sync_copy(srcr.at[wid].at[b], sidx)
        pltpu.sync_copy(dstr.at[wid].at[b], didx)
        pltpu.async_copy(table.at[sidx.at[0]], rows0, sem0)

        def pair(jj, inner):
            c0 = 2 * jj
            c1 = c0 + 1
            pltpu.async_copy(table.at[sidx.at[c1]], rows1, sem1)
            pltpu.make_async_copy(table.at[sidx.at[c0]], rows0, sem0).wait()
            pltpu.sync_copy(rows0, s_sh.at[didx.at[c0]], add=True)

            @pl.when(c0 + 2 < BLK_CH)
            def _():
                pltpu.async_copy(table.at[sidx.at[c0 + 2]], rows0, sem0)

            pltpu.make_async_copy(table.at[sidx.at[c1]], rows1, sem1).wait()
            pltpu.sync_copy(rows1, s_sh.at[didx.at[c1]], add=True)
            return inner

        lax.fori_loop(0, BLK_CH // 2, pair, carry)
        return carry

    lax.fori_loop(0, N_BLKS, block, 0)
    plsc.subcore_barrier()

    rs = pl.ds(s * ROWS_PER_TILE, ROWS_PER_TILE)
    pltpu.sync_copy(s_sh.at[rs], s_out.at[c].at[rs])


def _sc_deg_body(dstr, zrows, ones128,
                 deg_out,
                 didx, ones_v, deg_sh, sem):
    c = lax.axis_index("c")
    s = lax.axis_index("s")
    wid = s * 2 + c
    k = 8

    pltpu.sync_copy(zrows, deg_sh.at[pl.ds(s * ROWS_PER_TILE, ROWS_PER_TILE)])
    pltpu.sync_copy(ones128, ones_v)
    plsc.subcore_barrier()

    def block(b, carry):
        pltpu.sync_copy(dstr.at[wid].at[b], didx)

        def grp(g, inner):
            for i in range(k):
                pltpu.async_copy(ones_v, deg_sh.at[didx.at[g * k + i]], sem,
                                 add=True)
            for i in range(k):
                pltpu.make_async_copy(ones_v, deg_sh.at[didx.at[g * k + i]],
                                      sem).wait()
            return inner

        lax.fori_loop(0, BLK_CH // k, grp, carry)
        return carry

    lax.fori_loop(0, N_BLKS, block, 0)
    plsc.subcore_barrier()

    rs = pl.ds(s * ROWS_PER_TILE, ROWS_PER_TILE)
    pltpu.sync_copy(deg_sh.at[rs], deg_out.at[c].at[rs])


_sc_agg = pl.kernel(
    _sc_agg_body,
    out_type=(jax.ShapeDtypeStruct((2, NPAD, 128), jnp.float32),),
    mesh=plsc.VectorSubcoreMesh(**_MESH),
    scratch_types=(
        pltpu.VMEM((BLK_CH, CHUNK), jnp.int32),
        pltpu.VMEM((BLK_CH, CHUNK), jnp.int32),
        pltpu.VMEM((CHUNK, 128), jnp.float32),
        pltpu.VMEM((CHUNK, 128), jnp.float32),
        pltpu.VMEM_SHARED((NPAD, 128), jnp.float32),
        pltpu.SemaphoreType.DMA,
        pltpu.SemaphoreType.DMA,
    ),
)

_sc_deg = pl.kernel(
    _sc_deg_body,
    out_type=(jax.ShapeDtypeStruct((2, NPAD, 128), jnp.float32),),
    mesh=plsc.VectorSubcoreMesh(**_MESH),
    scratch_types=(
        pltpu.VMEM((BLK_CH, CHUNK), jnp.int32),
        pltpu.VMEM((CHUNK, 128), jnp.float32),
        pltpu.VMEM_SHARED((NPAD, 128), jnp.float32),
        pltpu.SemaphoreType.DMA,
    ),
)


_R = 1000


def _deg_factors(dref):
    d = dref[0][:, 0:1] + dref[1][:, 0:1]
    invd = 1.0 / jnp.maximum(d, 1.0)
    sself = 0.5 * d * invd
    hinv = 0.5 * invd
    hasdeg = (d > 0.0).astype(jnp.float32)
    return sself, hinv, hasdeg


def _t1_body(f, sa, dref, w, b, o):
    sself, hinv, hasdeg = _deg_factors(dref)
    p = sself * f[...] + hinv * (sa[0] + sa[1])
    o[...] = jax.nn.relu(
        jnp.dot(p, w[...], preferred_element_type=jnp.float32) + hasdeg * b[...])


def _t2_body(h, w, b, o):
    o[...] = jnp.dot(h[...], w[...], preferred_element_type=jnp.float32) + b[...]


def _t3_body(z, sb, dref, w, b, h2, z3):
    sself, hinv, _ = _deg_factors(dref)
    h = jax.nn.relu(sself * z[...] + hinv * (sb[0] + sb[1]))
    h2[...] = h
    z3[...] = jnp.dot(h, w[...], preferred_element_type=jnp.float32) + b[...]


def _t4_body(z, sc, dref, o):
    sself, hinv, _ = _deg_factors(dref)
    o[...] = jax.nn.relu(sself * z[...] + hinv * (sc[0] + sc[1]))


def _row_spec(cols):
    return pl.BlockSpec((_R, cols), lambda i: (i, 0))


def _part_spec(cols):
    return pl.BlockSpec((2, _R, cols), lambda i: (0, i, 0))


def _full_spec(r, c):
    return pl.BlockSpec((r, c), lambda i: (0, 0))


def _t1(f, sa, degp, w1, b1):
    return pl.pallas_call(
        _t1_body,
        grid=(N_NODES // _R,),
        in_specs=[_row_spec(IN_DIM), _part_spec(128), _part_spec(128),
                  _full_spec(IN_DIM, HID_DIM), _full_spec(1, HID_DIM)],
        out_specs=_row_spec(HID_DIM),
        out_shape=jax.ShapeDtypeStruct((N_NODES, HID_DIM), jnp.float32),
    )(f, sa, degp, w1, b1)


def _t2(h, w2, b2):
    return pl.pallas_call(
        _t2_body,
        grid=(N_NODES // _R,),
        in_specs=[_row_spec(HID_DIM), _full_spec(HID_DIM, OUT_DIM),
                  _full_spec(1, OUT_DIM)],
        out_specs=_row_spec(OUT_DIM),
        out_shape=jax.ShapeDtypeStruct((N_NODES, OUT_DIM), jnp.float32),
    )(h, w2, b2)


def _t3(z2, sb, degp, w3, b3):
    return pl.pallas_call(
        _t3_body,
        grid=(N_NODES // _R,),
        in_specs=[_row_spec(OUT_DIM), _part_spec(128), _part_spec(128),
                  _full_spec(OUT_DIM, OUT_DIM), _full_spec(1, OUT_DIM)],
        out_specs=[_row_spec(OUT_DIM), _row_spec(OUT_DIM)],
        out_shape=[jax.ShapeDtypeStruct((N_NODES, OUT_DIM), jnp.float32),
                   jax.ShapeDtypeStruct((N_NODES, OUT_DIM), jnp.float32)],
    )(z2, sb, degp, w3, b3)


def _t4(z3, sc, degp):
    return pl.pallas_call(
        _t4_body,
        grid=(N_NODES // _R,),
        in_specs=[_row_spec(OUT_DIM), _part_spec(128), _part_spec(128)],
        out_specs=_row_spec(OUT_DIM),
        out_shape=jax.ShapeDtypeStruct((N_NODES, OUT_DIM), jnp.float32),
    )(z3, sc, degp)


def kernel(features, edge_index, W1, b1, W2, b2, W3, b3):
    src = edge_index[0].astype(jnp.int32)
    dst = edge_index[1].astype(jnp.int32)
    src2 = jnp.concatenate([src, dst])
    dst2 = jnp.concatenate([dst, src])
    pad = E2_PAD - N_EDGES2
    srcr = jnp.pad(src2, (0, pad)).reshape(N_TILES, N_BLKS, BLK_CH, CHUNK)
    dstr = jnp.pad(dst2, (0, pad), constant_values=NPAD - 1) \
              .reshape(N_TILES, N_BLKS, BLK_CH, CHUNK)

    zrows = jnp.zeros((ROWS_PER_TILE, 128), jnp.float32)
    ones128 = jnp.ones((CHUNK, 128), jnp.float32)

    b1r = b1.reshape(1, HID_DIM)
    b2r = b2.reshape(1, OUT_DIM)
    b3r = b3.reshape(1, OUT_DIM)

    (degp,) = _sc_deg(dstr, zrows, ones128)
    (sa,) = _sc_agg(features, srcr, dstr, zrows)
    h1 = _t1(features, sa, degp, W1, b1r)
    z2 = _t2(h1, W2, b2r)
    (sb,) = _sc_agg(z2, srcr, dstr, zrows)
    h2, z3 = _t3(z2, sb, degp, W3, b3r)
    (sc,) = _sc_agg(z3, srcr, dstr, zrows)
    logits = _t4(z3, sc, degp)
    return (h2, logits)
